# baseline (speedup 1.0000x reference)
import jax
import jax.numpy as jnp
from jax.experimental import pallas as pl
from jax.experimental.pallas import tpu as pltpu


def _prologue_kernel(x_ref, wt_ref, r1_ref, r2_ref,
                     wh_ref, whb_ref, cs_ref, e1_ref, f1_ref, e2_ref, f2_ref):
    wh = jnp.dot(x_ref[...], wt_ref[...], preferred_element_type=jnp.float32)
    wh_ref[...] = wh
    whb_ref[...] = wh.astype(jnp.bfloat16)
    cs_ref[...] = jnp.sum(wh, axis=0, keepdims=True)
    s1 = jnp.dot(wh, r1_ref[...], preferred_element_type=jnp.float32)
    s2 = jnp.dot(wh, r2_ref[...], preferred_element_type=jnp.float32)
    e1_ref[...] = jnp.exp(s1)
    f1_ref[...] = jnp.exp(0.2 * s1)
    e2_ref[...] = jnp.exp(s2)
    f2_ref[...] = jnp.exp(0.2 * s2)


def _attn_kernel(a_ref, e1_ref, f1_ref, e2_ref, f2_ref, whb_ref, cs_ref,
                 out_ref):
    a = a_ref[...]
    p = jnp.maximum(e1_ref[...] * e2_ref[...], f1_ref[...] * f2_ref[...])
    pm1 = jnp.minimum(p, 1e38) - 1.0
    b = a * pm1
    den = jnp.sum(b, axis=1, keepdims=True) + jnp.sum(a, axis=1, keepdims=True)
    acc = cs_ref[...] + jnp.dot(b.astype(jnp.bfloat16), whb_ref[...],
                                preferred_element_type=jnp.float32)
    x = acc / den
    out_ref[...] = 0.5 * x * (1.0 + jax.lax.erf(x * 0.7071067811865476))


@jax.jit
def kernel(X, A, W, r):
    n, d_in = X.shape
    d_out = W.shape[0]

    bm = 1024

    vec = jax.ShapeDtypeStruct((n, 1), jnp.float32)
    wh, whb, cs, e1, f1, e2, f2 = pl.pallas_call(
        _prologue_kernel,
        grid=(1,),
        in_specs=[
            pl.BlockSpec((n, d_in), lambda i: (0, 0)),
            pl.BlockSpec((d_in, d_out), lambda i: (0, 0)),
            pl.BlockSpec((d_out, 1), lambda i: (0, 0)),
            pl.BlockSpec((d_out, 1), lambda i: (0, 0)),
        ],
        out_specs=[
            pl.BlockSpec((n, d_out), lambda i: (0, 0)),
            pl.BlockSpec((n, d_out), lambda i: (0, 0)),
            pl.BlockSpec((1, d_out), lambda i: (0, 0)),
            pl.BlockSpec((n, 1), lambda i: (0, 0)),
            pl.BlockSpec((n, 1), lambda i: (0, 0)),
            pl.BlockSpec((n, 1), lambda i: (0, 0)),
            pl.BlockSpec((n, 1), lambda i: (0, 0)),
        ],
        out_shape=[
            jax.ShapeDtypeStruct((n, d_out), jnp.float32),
            jax.ShapeDtypeStruct((n, d_out), jnp.bfloat16),
            jax.ShapeDtypeStruct((1, d_out), jnp.float32),
            vec, vec, vec, vec,
        ],
    )(X, W.T, r[:d_out], r[d_out:])

    e2r = e2.reshape(1, n)
    f2r = f2.reshape(1, n)

    out = pl.pallas_call(
        _attn_kernel,
        grid=(n // bm,),
        in_specs=[
            pl.BlockSpec((bm, n), lambda i: (i, 0)),
            pl.BlockSpec((bm, 1), lambda i: (i, 0)),
            pl.BlockSpec((bm, 1), lambda i: (i, 0)),
            pl.BlockSpec((1, n), lambda i: (0, 0)),
            pl.BlockSpec((1, n), lambda i: (0, 0)),
            pl.BlockSpec((n, d_out), lambda i: (0, 0)),
            pl.BlockSpec((1, d_out), lambda i: (0, 0)),
        ],
        out_specs=pl.BlockSpec((bm, d_out), lambda i: (i, 0)),
        out_shape=jax.ShapeDtypeStruct((n, d_out), jnp.float32),
        compiler_params=pltpu.CompilerParams(
            dimension_semantics=("arbitrary",),
        ),
    )(A, e1, f1, e2r, f2r, whb, cs)

    return out


# speedup vs baseline: 1.1181x; 1.1181x over previous
import jax
import jax.numpy as jnp
from jax.experimental import pallas as pl
from jax.experimental.pallas import tpu as pltpu


def _prologue_kernel(x_ref, wt_ref, r1_ref, r2_ref,
                     whext_ref, whb_ref, cs_ref, e1_ref, f1_ref, e2_ref, f2_ref):
    n = x_ref.shape[0]
    d = wt_ref.shape[1]
    wh = jnp.dot(x_ref[...], wt_ref[...], preferred_element_type=jnp.float32)
    whb = wh.astype(jnp.bfloat16)
    whb_ref[...] = whb
    lane = jax.lax.broadcasted_iota(jnp.int32, (n, d), 1)
    onescol = jnp.where(lane == 0, 1.0, 0.0).astype(jnp.bfloat16)
    whext_ref[...] = jnp.concatenate([whb, onescol], axis=1)
    cs_ref[...] = jnp.sum(wh, axis=0, keepdims=True)
    s1 = jnp.dot(wh, r1_ref[...], preferred_element_type=jnp.float32)
    s2 = jnp.dot(wh, r2_ref[...], preferred_element_type=jnp.float32)
    e1_ref[...] = jnp.exp(s1)
    f1_ref[...] = jnp.exp(0.2 * s1)
    e2_ref[...] = jnp.exp(s2)
    f2_ref[...] = jnp.exp(0.2 * s2)


def _attn_kernel(a_ref, e1_ref, f1_ref, e2_ref, f2_ref, whext_ref, whb_ref,
                 cs_ref, out_ref):
    d = whb_ref.shape[1]
    a = a_ref[...]
    p = jnp.maximum(e1_ref[...] * e2_ref[...], f1_ref[...] * f2_ref[...])
    m = (a * jnp.minimum(p, 1e38)).astype(jnp.bfloat16)
    ab = a.astype(jnp.bfloat16)
    r1 = jnp.dot(m, whext_ref[...], preferred_element_type=jnp.float32)
    r2 = jnp.dot(ab, whb_ref[...], preferred_element_type=jnp.float32)
    acc = r1[:, :d] - r2 + cs_ref[...]
    den = r1[:, d:d + 1]
    x = acc / den
    out_ref[...] = 0.5 * x * (1.0 + jax.lax.erf(x * 0.7071067811865476))


@jax.jit
def kernel(X, A, W, r):
    n, d_in = X.shape
    d_out = W.shape[0]

    bm = 512

    vec = jax.ShapeDtypeStruct((n, 1), jnp.float32)
    whext, whb, cs, e1, f1, e2, f2 = pl.pallas_call(
        _prologue_kernel,
        grid=(1,),
        in_specs=[
            pl.BlockSpec((n, d_in), lambda i: (0, 0)),
            pl.BlockSpec((d_in, d_out), lambda i: (0, 0)),
            pl.BlockSpec((d_out, 1), lambda i: (0, 0)),
            pl.BlockSpec((d_out, 1), lambda i: (0, 0)),
        ],
        out_specs=[
            pl.BlockSpec((n, 2 * d_out), lambda i: (0, 0)),
            pl.BlockSpec((n, d_out), lambda i: (0, 0)),
            pl.BlockSpec((1, d_out), lambda i: (0, 0)),
            pl.BlockSpec((n, 1), lambda i: (0, 0)),
            pl.BlockSpec((n, 1), lambda i: (0, 0)),
            pl.BlockSpec((n, 1), lambda i: (0, 0)),
            pl.BlockSpec((n, 1), lambda i: (0, 0)),
        ],
        out_shape=[
            jax.ShapeDtypeStruct((n, 2 * d_out), jnp.bfloat16),
            jax.ShapeDtypeStruct((n, d_out), jnp.bfloat16),
            jax.ShapeDtypeStruct((1, d_out), jnp.float32),
            vec, vec, vec, vec,
        ],
    )(X, W.T, r[:d_out], r[d_out:])

    e2r = e2.reshape(1, n)
    f2r = f2.reshape(1, n)

    out = pl.pallas_call(
        _attn_kernel,
        grid=(n // bm,),
        in_specs=[
            pl.BlockSpec((bm, n), lambda i: (i, 0)),
            pl.BlockSpec((bm, 1), lambda i: (i, 0)),
            pl.BlockSpec((bm, 1), lambda i: (i, 0)),
            pl.BlockSpec((1, n), lambda i: (0, 0)),
            pl.BlockSpec((1, n), lambda i: (0, 0)),
            pl.BlockSpec((n, 2 * d_out), lambda i: (0, 0)),
            pl.BlockSpec((n, d_out), lambda i: (0, 0)),
            pl.BlockSpec((1, d_out), lambda i: (0, 0)),
        ],
        out_specs=pl.BlockSpec((bm, d_out), lambda i: (i, 0)),
        out_shape=jax.ShapeDtypeStruct((n, d_out), jnp.float32),
        compiler_params=pltpu.CompilerParams(
            dimension_semantics=("arbitrary",),
        ),
    )(A, e1, f1, e2r, f2r, whext, whb, cs)

    return out


# speedup vs baseline: 1.1209x; 1.0025x over previous
import jax
import jax.numpy as jnp
from jax.experimental import pallas as pl
from jax.experimental.pallas import tpu as pltpu


def _prologue_kernel(x_ref, wt_ref, r1_ref, r2_ref,
                     whext_ref, whb_ref, cs_ref, e1_ref, g1_ref, e2_ref, f2_ref):
    n = x_ref.shape[0]
    d = wt_ref.shape[1]
    wh = jnp.dot(x_ref[...], wt_ref[...], preferred_element_type=jnp.float32)
    whb = wh.astype(jnp.bfloat16)
    whb_ref[...] = whb
    lane = jax.lax.broadcasted_iota(jnp.int32, (n, d), 1)
    onescol = jnp.where(lane == 0, 1.0, 0.0).astype(jnp.bfloat16)
    whext_ref[...] = jnp.concatenate([whb, onescol], axis=1)
    cs_ref[...] = jnp.sum(wh, axis=0, keepdims=True)
    s1 = jnp.dot(wh, r1_ref[...], preferred_element_type=jnp.float32)
    s2 = jnp.dot(wh, r2_ref[...], preferred_element_type=jnp.float32)
    e1_ref[...] = jnp.exp(s1)
    g1_ref[...] = jnp.exp(-0.8 * s1)
    e2_ref[...] = jnp.exp(s2)
    f2_ref[...] = jnp.exp(0.2 * s2)


def _attn_kernel(a_ref, e1_ref, g1_ref, e2_ref, f2_ref, whext_ref, whb_ref,
                 cs_ref, out_ref):
    d = whb_ref.shape[1]
    a = a_ref[...]
    t = jnp.maximum(e2_ref[...], g1_ref[...] * f2_ref[...])
    m = (a * t).astype(jnp.bfloat16)
    ab = a.astype(jnp.bfloat16)
    r1 = jnp.dot(m, whext_ref[...], preferred_element_type=jnp.float32)
    r2 = jnp.dot(ab, whb_ref[...], preferred_element_type=jnp.float32)
    e1 = e1_ref[...]
    acc = e1 * r1[:, :d] - r2 + cs_ref[...]
    den = e1 * r1[:, d:d + 1]
    x = acc / den
    out_ref[...] = 0.5 * x * (1.0 + jax.lax.erf(x * 0.7071067811865476))


@jax.jit
def kernel(X, A, W, r):
    n, d_in = X.shape
    d_out = W.shape[0]

    bm = 512

    vec = jax.ShapeDtypeStruct((n, 1), jnp.float32)
    whext, whb, cs, e1, g1, e2, f2 = pl.pallas_call(
        _prologue_kernel,
        grid=(1,),
        in_specs=[
            pl.BlockSpec((n, d_in), lambda i: (0, 0)),
            pl.BlockSpec((d_in, d_out), lambda i: (0, 0)),
            pl.BlockSpec((d_out, 1), lambda i: (0, 0)),
            pl.BlockSpec((d_out, 1), lambda i: (0, 0)),
        ],
        out_specs=[
            pl.BlockSpec((n, 2 * d_out), lambda i: (0, 0)),
            pl.BlockSpec((n, d_out), lambda i: (0, 0)),
            pl.BlockSpec((1, d_out), lambda i: (0, 0)),
            pl.BlockSpec((n, 1), lambda i: (0, 0)),
            pl.BlockSpec((n, 1), lambda i: (0, 0)),
            pl.BlockSpec((n, 1), lambda i: (0, 0)),
            pl.BlockSpec((n, 1), lambda i: (0, 0)),
        ],
        out_shape=[
            jax.ShapeDtypeStruct((n, 2 * d_out), jnp.bfloat16),
            jax.ShapeDtypeStruct((n, d_out), jnp.bfloat16),
            jax.ShapeDtypeStruct((1, d_out), jnp.float32),
            vec, vec, vec, vec,
        ],
    )(X, W.T, r[:d_out], r[d_out:])

    e2r = e2.reshape(1, n)
    f2r = f2.reshape(1, n)

    out = pl.pallas_call(
        _attn_kernel,
        grid=(n // bm,),
        in_specs=[
            pl.BlockSpec((bm, n), lambda i: (i, 0)),
            pl.BlockSpec((bm, 1), lambda i: (i, 0)),
            pl.BlockSpec((bm, 1), lambda i: (i, 0)),
            pl.BlockSpec((1, n), lambda i: (0, 0)),
            pl.BlockSpec((1, n), lambda i: (0, 0)),
            pl.BlockSpec((n, 2 * d_out), lambda i: (0, 0)),
            pl.BlockSpec((n, d_out), lambda i: (0, 0)),
            pl.BlockSpec((1, d_out), lambda i: (0, 0)),
        ],
        out_specs=pl.BlockSpec((bm, d_out), lambda i: (i, 0)),
        out_shape=jax.ShapeDtypeStruct((n, d_out), jnp.float32),
        compiler_params=pltpu.CompilerParams(
            dimension_semantics=("arbitrary",),
        ),
    )(A, e1, g1, e2r, f2r, whext, whb, cs)

    return out
